# baseline (speedup 1.0000x reference)
import functools

import jax
import jax.numpy as jnp
import numpy as np
from jax import lax
from jax.experimental import pallas as pl
from jax.experimental.pallas import tpu as pltpu
from jax.experimental.pallas import tpu_sc as plsc

N = 10000
E = 160000
F = 128
R = 20
TF = 3 * F

NC = 2
NS = 16
L = 16

E_PAD = 163840
EDGES_PER_TILE = E_PAD // (NC * NS)
N_PAD = 10240
ROWS_PER_TILE = N_PAD // NS
CHUNK = 32

SBATCH = 80
VBATCH = 32

SW = 128
PW = 256


def _phi_body(x_ref, w1t_ref, b1_ref, w2t_ref, b2_ref, os_ref, ov_ref):
    x = x_ref[...]
    h = x @ w1t_ref[...] + b1_ref[...]
    h = h * jax.nn.sigmoid(h)
    p = h @ w2t_ref[...] + b2_ref[...]
    os_ref[...] = p[:, :SW]
    ov_ref[...] = p[:, SW:]


def _phi_tables(x, w1t, b1, w2t, b2p):
    bm = 400
    grid = (N // bm,)
    return pl.pallas_call(
        _phi_body,
        grid=grid,
        in_specs=[
            pl.BlockSpec((bm, F), lambda i: (i, 0)),
            pl.BlockSpec((F, F), lambda i: (0, 0)),
            pl.BlockSpec((1, F), lambda i: (0, 0)),
            pl.BlockSpec((F, TF), lambda i: (0, 0)),
            pl.BlockSpec((1, TF), lambda i: (0, 0)),
        ],
        out_specs=[
            pl.BlockSpec((bm, SW), lambda i: (i, 0)),
            pl.BlockSpec((bm, PW), lambda i: (i, 0)),
        ],
        out_shape=[
            jax.ShapeDtypeStruct((N, SW), jnp.float32),
            jax.ShapeDtypeStruct((N, PW), jnp.float32),
        ],
    )(x, w1t, b1, w2t, b2p)


def _wedge_body(rbf_ref, wrt_ref, br_ref, rdc_ref, ws_ref, wv_ref):
    p = (rbf_ref[...] @ wrt_ref[...] + br_ref[...]) * rdc_ref[...]
    ws_ref[...] = p[:, :SW]
    wv_ref[...] = p[:, SW:]


def _wedge_tables(rbf, wrt, brp, rdc2d):
    bm = 4096
    grid = (E_PAD // bm,)
    return pl.pallas_call(
        _wedge_body,
        grid=grid,
        in_specs=[
            pl.BlockSpec((bm, R), lambda i: (i, 0)),
            pl.BlockSpec((R, TF), lambda i: (0, 0)),
            pl.BlockSpec((1, TF), lambda i: (0, 0)),
            pl.BlockSpec((bm, 1), lambda i: (i, 0)),
        ],
        out_specs=[
            pl.BlockSpec((bm, SW), lambda i: (i, 0)),
            pl.BlockSpec((bm, PW), lambda i: (i, 0)),
        ],
        out_shape=[
            jax.ShapeDtypeStruct((E_PAD, SW), jnp.float32),
            jax.ShapeDtypeStruct((E_PAD, PW), jnp.float32),
        ],
    )(rbf, wrt, brp, rdc2d)


def _lane_bcast(ref, e):
    v = ref[pl.ds(e, L)]
    z = (v * 0.0).astype(jnp.int32)
    return v.at[z].get(mode="promise_in_bounds")


def _zero_and_fill(z_v, acc, r0):
    def zrow(i, carry):
        for kk in range(SW // L):
            z_v[i, pl.ds(kk * L, L)] = jnp.zeros((L,), jnp.float32)
        return carry

    lax.fori_loop(0, CHUNK, zrow, 0)

    def fill(i, carry):
        pltpu.sync_copy(z_v.at[pl.ds(0, CHUNK)],
                        acc.at[pl.ds(r0 + i * CHUNK, CHUNK)])
        return carry

    lax.fori_loop(0, ROWS_PER_TILE // CHUNK, fill, 0)


def _flush_slot(acc, out_hbm, slot, c, r0, bounce):
    def body(i, carry):
        sl = pl.ds(r0 + i * CHUNK, CHUNK)
        pltpu.sync_copy(acc.at[sl], bounce.at[pl.ds(0, CHUNK)])
        pltpu.sync_copy(bounce.at[pl.ds(0, CHUNK)], out_hbm.at[slot, c, sl])
        return carry

    lax.fori_loop(0, ROWS_PER_TILE // CHUNK, body, 0)


def _flush(acc, out_hbm, c, r0, bounce):
    def body(i, carry):
        sl = pl.ds(r0 + i * CHUNK, CHUNK)
        pltpu.sync_copy(acc.at[sl], bounce.at[pl.ds(0, CHUNK)])
        pltpu.sync_copy(bounce.at[pl.ds(0, CHUNK)], out_hbm.at[c, sl])
        return carry

    lax.fori_loop(0, ROWS_PER_TILE // CHUNK, body, 0)


def _make_scalar_sc():
    mesh = plsc.VectorSubcoreMesh(core_axis_name="c", subcore_axis_name="s",
                                  num_cores=NC, num_subcores=NS)
    nb = EDGES_PER_TILE // SBATCH

    @functools.partial(
        pl.kernel,
        out_type=jax.ShapeDtypeStruct((NC, N_PAD, SW), jnp.float32),
        mesh=mesh,
        scratch_types=[
            pltpu.VMEM_SHARED((N_PAD, SW), jnp.float32),
            pltpu.SemaphoreType.DMA,
            pltpu.SemaphoreType.DMA,
            pltpu.SemaphoreType.DMA,
        ],
    )
    def k(idx_i_hbm, idx_j_hbm, phi_hbm, w_hbm, out_hbm, acc,
          semL, semG0, semG1):
        c = lax.axis_index("c")
        s = lax.axis_index("s")
        semG = [semG0, semG1]

        def run(*bufs):
            sets = [bufs[:4], bufs[4:]]
            r0 = s * ROWS_PER_TILE
            _zero_and_fill(sets[0][3], acc, r0)
            plsc.subcore_barrier()

            tile_base = (c * NS + s) * EDGES_PER_TILE

            def issue_linear(b, si):
                idxj_v, idxi_v, w_v, pj_v = sets[si]
                bb = jnp.minimum(b, nb - 1)
                base = tile_base + bb * SBATCH
                pltpu.async_copy(idx_j_hbm.at[pl.ds(base, SBATCH)],
                                 idxj_v, semL)
                pltpu.async_copy(idx_i_hbm.at[pl.ds(base, SBATCH)],
                                 idxi_v, semL)
                pltpu.async_copy(w_hbm.at[pl.ds(base, SBATCH)], w_v, semL)

            def wait_linear(si):
                idxj_v, idxi_v, w_v, pj_v = sets[si]
                pltpu.make_async_copy(
                    idx_j_hbm.at[pl.ds(0, SBATCH)], idxj_v, semL).wait()
                pltpu.make_async_copy(
                    idx_i_hbm.at[pl.ds(0, SBATCH)], idxi_v, semL).wait()
                pltpu.make_async_copy(
                    w_hbm.at[pl.ds(0, SBATCH)], w_v, semL).wait()

            def issue_gather(si):
                idxj_v, idxi_v, w_v, pj_v = sets[si]
                pltpu.async_copy(phi_hbm.at[idxj_v], pj_v, semG[si])

            def wait_gather(si):
                idxj_v, idxi_v, w_v, pj_v = sets[si]
                pltpu.make_async_copy(phi_hbm.at[idxj_v], pj_v,
                                      semG[si]).wait()

            def compute_scatter(si):
                idxj_v, idxi_v, w_v, pj_v = sets[si]

                def edge_body(e, c2):
                    for kk in range(SW // L):
                        sl = pl.ds(kk * L, L)
                        pj_v[e, sl] = pj_v[e, sl] * w_v[e, sl]
                    return c2

                lax.fori_loop(0, SBATCH, edge_body, 0)
                pltpu.sync_copy(pj_v, acc.at[idxi_v], add=True)

            issue_linear(0, 0)
            wait_linear(0)
            issue_gather(0)
            issue_linear(1, 1)

            def pair_body(t, carry):
                b0 = 2 * t
                wait_linear(1)
                issue_gather(1)
                wait_gather(0)
                compute_scatter(0)
                issue_linear(b0 + 2, 0)
                wait_linear(0)
                issue_gather(0)
                wait_gather(1)
                compute_scatter(1)
                issue_linear(b0 + 3, 1)
                return carry

            lax.fori_loop(0, nb // 2, pair_body, 0)
            wait_gather(0)
            wait_linear(1)

            plsc.subcore_barrier()
            _flush(acc, out_hbm, c, r0, sets[0][3])

        pl.run_scoped(run,
                      pltpu.VMEM((SBATCH,), jnp.int32),
                      pltpu.VMEM((SBATCH,), jnp.int32),
                      pltpu.VMEM((SBATCH, SW), jnp.float32),
                      pltpu.VMEM((SBATCH, SW), jnp.float32),
                      pltpu.VMEM((SBATCH,), jnp.int32),
                      pltpu.VMEM((SBATCH,), jnp.int32),
                      pltpu.VMEM((SBATCH, SW), jnp.float32),
                      pltpu.VMEM((SBATCH, SW), jnp.float32))

    return k


def _make_dims_sc():
    mesh = plsc.VectorSubcoreMesh(core_axis_name="c", subcore_axis_name="s",
                                  num_cores=NC, num_subcores=NS)
    nb = EDGES_PER_TILE // VBATCH

    @functools.partial(
        pl.kernel,
        out_type=jax.ShapeDtypeStruct((3, NC, N_PAD, SW), jnp.float32),
        mesh=mesh,
        scratch_types=[
            pltpu.VMEM_SHARED((N_PAD, SW), jnp.float32),
            pltpu.SemaphoreType.DMA,
            pltpu.SemaphoreType.DMA,
            pltpu.SemaphoreType.DMA,
        ],
    )
    def k(idx_i_hbm, idx_j_hbm, phi_hbm, w_hbm,
          vfd0_hbm, vfd1_hbm, vfd2_hbm, d0_hbm, d1_hbm, d2_hbm,
          out_hbm, acc, semL, semG0, semG1):
        c = lax.axis_index("c")
        s = lax.axis_index("s")
        semG = [semG0, semG1]
        vfd_hbms = [vfd0_hbm, vfd1_hbm, vfd2_hbm]
        dd_hbms = [d0_hbm, d1_hbm, d2_hbm]

        def run(*bufs):
            sets = [bufs[:6], bufs[6:]]
            r0 = s * ROWS_PER_TILE
            tile_base = (c * NS + s) * EDGES_PER_TILE

            def phase(vfd_hbm, dd_hbm, slot):
                _zero_and_fill(sets[0][4], acc, r0)
                plsc.subcore_barrier()

                def issue_linear(b, si):
                    idxj_v, idxi_v, w_v, pj_v, vf_v, dd_v = sets[si]
                    bb = jnp.minimum(b, nb - 1)
                    base = tile_base + bb * VBATCH
                    pltpu.async_copy(idx_j_hbm.at[pl.ds(base, VBATCH)],
                                     idxj_v, semL)
                    pltpu.async_copy(idx_i_hbm.at[pl.ds(base, VBATCH)],
                                     idxi_v, semL)
                    pltpu.async_copy(w_hbm.at[pl.ds(base, VBATCH)], w_v, semL)
                    pltpu.async_copy(dd_hbm.at[pl.ds(base, VBATCH)],
                                     dd_v.at[pl.ds(0, VBATCH)], semL)

                def wait_linear(si):
                    idxj_v, idxi_v, w_v, pj_v, vf_v, dd_v = sets[si]
                    pltpu.make_async_copy(
                        idx_j_hbm.at[pl.ds(0, VBATCH)], idxj_v, semL).wait()
                    pltpu.make_async_copy(
                        idx_i_hbm.at[pl.ds(0, VBATCH)], idxi_v, semL).wait()
                    pltpu.make_async_copy(
                        w_hbm.at[pl.ds(0, VBATCH)], w_v, semL).wait()
                    pltpu.make_async_copy(
                        dd_hbm.at[pl.ds(0, VBATCH)],
                        dd_v.at[pl.ds(0, VBATCH)], semL).wait()

                def issue_gather(si):
                    idxj_v, idxi_v, w_v, pj_v, vf_v, dd_v = sets[si]
                    pltpu.async_copy(phi_hbm.at[idxj_v], pj_v, semG[si])
                    pltpu.async_copy(vfd_hbm.at[idxj_v], vf_v, semG[si])

                def wait_gather(si):
                    idxj_v, idxi_v, w_v, pj_v, vf_v, dd_v = sets[si]
                    pltpu.make_async_copy(phi_hbm.at[idxj_v], pj_v,
                                          semG[si]).wait()
                    pltpu.make_async_copy(vfd_hbm.at[idxj_v], vf_v,
                                          semG[si]).wait()

                def compute_scatter(si):
                    idxj_v, idxi_v, w_v, pj_v, vf_v, dd_v = sets[si]

                    def edge_body(e, c2):
                        db = _lane_bcast(dd_v, e)
                        for kk in range(SW // L):
                            sl = pl.ds(kk * L, L)
                            sv = pl.ds(SW + kk * L, L)
                            vvw = pj_v[e, sl] * w_v[e, sl]
                            vsw = pj_v[e, sv] * w_v[e, sv]
                            vf_v[e, sl] = vf_v[e, sl] * vvw + vsw * db
                        return c2

                    lax.fori_loop(0, VBATCH, edge_body, 0)
                    pltpu.sync_copy(vf_v, acc.at[idxi_v], add=True)

                issue_linear(0, 0)
                wait_linear(0)
                issue_gather(0)
                issue_linear(1, 1)

                def pair_body(t, carry):
                    b0 = 2 * t
                    wait_linear(1)
                    issue_gather(1)
                    wait_gather(0)
                    compute_scatter(0)
                    issue_linear(b0 + 2, 0)
                    wait_linear(0)
                    issue_gather(0)
                    wait_gather(1)
                    compute_scatter(1)
                    issue_linear(b0 + 3, 1)
                    return carry

                lax.fori_loop(0, nb // 2, pair_body, 0)
                wait_gather(0)
                wait_linear(1)

                plsc.subcore_barrier()
                _flush_slot(acc, out_hbm, slot, c, r0, sets[0][4])

            phase(vfd_hbms[0], dd_hbms[0], 0)
            plsc.subcore_barrier()
            phase(vfd_hbms[1], dd_hbms[1], 1)
            plsc.subcore_barrier()
            phase(vfd_hbms[2], dd_hbms[2], 2)

        pl.run_scoped(run,
                      pltpu.VMEM((VBATCH,), jnp.int32),
                      pltpu.VMEM((VBATCH,), jnp.int32),
                      pltpu.VMEM((VBATCH, PW), jnp.float32),
                      pltpu.VMEM((VBATCH, PW), jnp.float32),
                      pltpu.VMEM((VBATCH, SW), jnp.float32),
                      pltpu.VMEM((VBATCH + L,), jnp.float32),
                      pltpu.VMEM((VBATCH,), jnp.int32),
                      pltpu.VMEM((VBATCH,), jnp.int32),
                      pltpu.VMEM((VBATCH, PW), jnp.float32),
                      pltpu.VMEM((VBATCH, PW), jnp.float32),
                      pltpu.VMEM((VBATCH, SW), jnp.float32),
                      pltpu.VMEM((VBATCH + L,), jnp.float32))

    return k


def _combine_body(sf_ref, vfd_ref, ps_ref, pd_ref, os_ref, ov_ref):
    os_ref[...] = sf_ref[...] + ps_ref[0] + ps_ref[1]
    ov_ref[0] = vfd_ref[0] + pd_ref[0, 0] + pd_ref[0, 1]
    ov_ref[1] = vfd_ref[1] + pd_ref[1, 0] + pd_ref[1, 1]
    ov_ref[2] = vfd_ref[2] + pd_ref[2, 0] + pd_ref[2, 1]


def _combine(sf, vfd, ps, pd):
    bm = 400
    grid = (N // bm,)
    return pl.pallas_call(
        _combine_body,
        grid=grid,
        in_specs=[
            pl.BlockSpec((bm, F), lambda i: (i, 0)),
            pl.BlockSpec((3, bm, SW), lambda i: (0, i, 0)),
            pl.BlockSpec((NC, bm, SW), lambda i: (0, i, 0)),
            pl.BlockSpec((3, NC, bm, SW), lambda i: (0, 0, i, 0)),
        ],
        out_specs=[
            pl.BlockSpec((bm, F), lambda i: (i, 0)),
            pl.BlockSpec((3, bm, SW), lambda i: (0, i, 0)),
        ],
        out_shape=[
            jax.ShapeDtypeStruct((N, F), jnp.float32),
            jax.ShapeDtypeStruct((3, N, SW), jnp.float32),
        ],
    )(sf, vfd, ps, pd)


_scalar_sc = _make_scalar_sc()
_dims_sc = _make_dims_sc()

_PERM = np.concatenate([
    np.arange(F, 2 * F),
    np.arange(0, F),
    np.arange(2 * F, 3 * F),
]).astype(np.int32)


def _pad_e(x):
    pad = [(0, E_PAD - E)] + [(0, 0)] * (x.ndim - 1)
    return jnp.pad(x, pad)


def kernel(idx_i, idx_j, rel_dir, rel_dist_cut, rbf_features, scalar_features,
           vector_features, W1, b1, W2, b2, Wr, br):
    idx_i = _pad_e(idx_i.astype(jnp.int32))
    idx_j = _pad_e(idx_j.astype(jnp.int32))

    w1t = W1.T
    w2t = W2[_PERM].T
    b2p = b2[_PERM].reshape(1, TF)
    wrt = Wr[_PERM].T
    brp = br[_PERM].reshape(1, TF)
    b1r = b1.reshape(1, F)
    rdc2d = _pad_e(rel_dist_cut.reshape(E, 1))

    phi_s, phi_v = _phi_tables(scalar_features, w1t, b1r, w2t, b2p)
    w_s, w_v = _wedge_tables(_pad_e(rbf_features), wrt, brp, rdc2d)

    vfd = jnp.transpose(vector_features, (2, 0, 1))
    rdp = _pad_e(rel_dir)

    ps = _scalar_sc(idx_i, idx_j, phi_s, w_s)
    pd = _dims_sc(idx_i, idx_j, phi_v, w_v, vfd[0], vfd[1], vfd[2],
                  rdp[:, 0], rdp[:, 1], rdp[:, 2])

    out_s, out_vd = _combine(scalar_features, vfd, ps, pd)
    return out_s, jnp.transpose(out_vd, (1, 2, 0))


# speedup vs baseline: 1.0308x; 1.0308x over previous
import functools

import jax
import jax.numpy as jnp
import numpy as np
from jax import lax
from jax.experimental import pallas as pl
from jax.experimental.pallas import tpu as pltpu
from jax.experimental.pallas import tpu_sc as plsc

N = 10000
E = 160000
F = 128
R = 20
TF = 3 * F

NC = 2
NS = 16
L = 16

E_PAD = 163840
EDGES_PER_TILE = E_PAD // (NC * NS)
N_PAD = 10240
ROWS_PER_TILE = N_PAD // NS
CHUNK = 32

SBATCH = 80
VBATCH = 32

SW = 128
PW = 256


def _phi_body(x_ref, w1t_ref, b1_ref, w2t_ref, b2_ref, os_ref, ov_ref):
    x = x_ref[...]
    h = x @ w1t_ref[...] + b1_ref[...]
    h = h * jax.nn.sigmoid(h)
    p = h @ w2t_ref[...] + b2_ref[...]
    os_ref[...] = p[:, :SW]
    ov_ref[...] = p[:, SW:]


def _phi_tables(x, w1t, b1, w2t, b2p):
    bm = 400
    grid = (N // bm,)
    return pl.pallas_call(
        _phi_body,
        grid=grid,
        in_specs=[
            pl.BlockSpec((bm, F), lambda i: (i, 0)),
            pl.BlockSpec((F, F), lambda i: (0, 0)),
            pl.BlockSpec((1, F), lambda i: (0, 0)),
            pl.BlockSpec((F, TF), lambda i: (0, 0)),
            pl.BlockSpec((1, TF), lambda i: (0, 0)),
        ],
        out_specs=[
            pl.BlockSpec((bm, SW), lambda i: (i, 0)),
            pl.BlockSpec((bm, PW), lambda i: (i, 0)),
        ],
        out_shape=[
            jax.ShapeDtypeStruct((N, SW), jnp.float32),
            jax.ShapeDtypeStruct((N, PW), jnp.float32),
        ],
    )(x, w1t, b1, w2t, b2p)


def _wedge_body(rbf_ref, wrt_ref, br_ref, rdc_ref, ws_ref, wv_ref):
    p = (rbf_ref[...] @ wrt_ref[...] + br_ref[...]) * rdc_ref[...]
    ws_ref[...] = p[:, :SW]
    wv_ref[...] = p[:, SW:]


def _wedge_tables(rbf, wrt, brp, rdc2d):
    bm = 4096
    grid = (E_PAD // bm,)
    return pl.pallas_call(
        _wedge_body,
        grid=grid,
        in_specs=[
            pl.BlockSpec((bm, R), lambda i: (i, 0)),
            pl.BlockSpec((R, TF), lambda i: (0, 0)),
            pl.BlockSpec((1, TF), lambda i: (0, 0)),
            pl.BlockSpec((bm, 1), lambda i: (i, 0)),
        ],
        out_specs=[
            pl.BlockSpec((bm, SW), lambda i: (i, 0)),
            pl.BlockSpec((bm, PW), lambda i: (i, 0)),
        ],
        out_shape=[
            jax.ShapeDtypeStruct((E_PAD, SW), jnp.float32),
            jax.ShapeDtypeStruct((E_PAD, PW), jnp.float32),
        ],
    )(rbf, wrt, brp, rdc2d)


def _lane_bcast(ref, e):
    v = ref[pl.ds(e, L)]
    z = (v * 0.0).astype(jnp.int32)
    return v.at[z].get(mode="promise_in_bounds")


def _zero_and_fill(z_v, acc, r0):
    def zrow(i, carry):
        for kk in range(SW // L):
            z_v[i, pl.ds(kk * L, L)] = jnp.zeros((L,), jnp.float32)
        return carry

    lax.fori_loop(0, CHUNK, zrow, 0)

    def fill(i, carry):
        pltpu.sync_copy(z_v.at[pl.ds(0, CHUNK)],
                        acc.at[pl.ds(r0 + i * CHUNK, CHUNK)])
        return carry

    lax.fori_loop(0, ROWS_PER_TILE // CHUNK, fill, 0)


def _flush_slot(acc, out_hbm, slot, c, r0, bounce):
    def body(i, carry):
        sl = pl.ds(r0 + i * CHUNK, CHUNK)
        pltpu.sync_copy(acc.at[sl], bounce.at[pl.ds(0, CHUNK)])
        pltpu.sync_copy(bounce.at[pl.ds(0, CHUNK)], out_hbm.at[slot, c, sl])
        return carry

    lax.fori_loop(0, ROWS_PER_TILE // CHUNK, body, 0)


def _flush(acc, out_hbm, c, r0, bounce):
    def body(i, carry):
        sl = pl.ds(r0 + i * CHUNK, CHUNK)
        pltpu.sync_copy(acc.at[sl], bounce.at[pl.ds(0, CHUNK)])
        pltpu.sync_copy(bounce.at[pl.ds(0, CHUNK)], out_hbm.at[c, sl])
        return carry

    lax.fori_loop(0, ROWS_PER_TILE // CHUNK, body, 0)


def _make_scalar_sc():
    mesh = plsc.VectorSubcoreMesh(core_axis_name="c", subcore_axis_name="s",
                                  num_cores=NC, num_subcores=NS)
    nb = EDGES_PER_TILE // SBATCH

    @functools.partial(
        pl.kernel,
        out_type=jax.ShapeDtypeStruct((NC, N_PAD, SW), jnp.float32),
        mesh=mesh,
        scratch_types=[
            pltpu.VMEM_SHARED((N_PAD, SW), jnp.float32),
            pltpu.SemaphoreType.DMA,
            pltpu.SemaphoreType.DMA,
            pltpu.SemaphoreType.DMA,
        ],
    )
    def k(idx_i_hbm, idx_j_hbm, phi_hbm, w_hbm, out_hbm, acc,
          semL, semG0, semG1):
        c = lax.axis_index("c")
        s = lax.axis_index("s")
        semG = [semG0, semG1]

        def run(*bufs):
            sets = [bufs[:4], bufs[4:]]
            r0 = s * ROWS_PER_TILE
            _zero_and_fill(sets[0][3], acc, r0)
            plsc.subcore_barrier()

            tile_base = (c * NS + s) * EDGES_PER_TILE

            def issue_linear(b, si):
                idxj_v, idxi_v, w_v, pj_v = sets[si]
                bb = jnp.minimum(b, nb - 1)
                base = tile_base + bb * SBATCH
                pltpu.async_copy(idx_j_hbm.at[pl.ds(base, SBATCH)],
                                 idxj_v, semL)
                pltpu.async_copy(idx_i_hbm.at[pl.ds(base, SBATCH)],
                                 idxi_v, semL)
                pltpu.async_copy(w_hbm.at[pl.ds(base, SBATCH)], w_v, semL)

            def wait_linear(si):
                idxj_v, idxi_v, w_v, pj_v = sets[si]
                pltpu.make_async_copy(
                    idx_j_hbm.at[pl.ds(0, SBATCH)], idxj_v, semL).wait()
                pltpu.make_async_copy(
                    idx_i_hbm.at[pl.ds(0, SBATCH)], idxi_v, semL).wait()
                pltpu.make_async_copy(
                    w_hbm.at[pl.ds(0, SBATCH)], w_v, semL).wait()

            def issue_gather(si):
                idxj_v, idxi_v, w_v, pj_v = sets[si]
                pltpu.async_copy(phi_hbm.at[idxj_v], pj_v, semG[si])

            def wait_gather(si):
                idxj_v, idxi_v, w_v, pj_v = sets[si]
                pltpu.make_async_copy(phi_hbm.at[idxj_v], pj_v,
                                      semG[si]).wait()

            def compute_scatter(si):
                idxj_v, idxi_v, w_v, pj_v = sets[si]

                def edge_body(e, c2):
                    for kk in range(SW // L):
                        sl = pl.ds(kk * L, L)
                        pj_v[e, sl] = pj_v[e, sl] * w_v[e, sl]
                    return c2

                lax.fori_loop(0, SBATCH, edge_body, 0)
                pltpu.sync_copy(pj_v, acc.at[idxi_v], add=True)

            issue_linear(0, 0)
            wait_linear(0)
            issue_gather(0)
            issue_linear(1, 1)

            def pair_body(t, carry):
                b0 = 2 * t
                wait_linear(1)
                issue_gather(1)
                wait_gather(0)
                compute_scatter(0)
                issue_linear(b0 + 2, 0)
                wait_linear(0)
                issue_gather(0)
                wait_gather(1)
                compute_scatter(1)
                issue_linear(b0 + 3, 1)
                return carry

            lax.fori_loop(0, nb // 2, pair_body, 0)
            wait_gather(0)
            wait_linear(1)

            plsc.subcore_barrier()
            _flush(acc, out_hbm, c, r0, sets[0][3])

        pl.run_scoped(run,
                      pltpu.VMEM((SBATCH,), jnp.int32),
                      pltpu.VMEM((SBATCH,), jnp.int32),
                      pltpu.VMEM((SBATCH, SW), jnp.float32),
                      pltpu.VMEM((SBATCH, SW), jnp.float32),
                      pltpu.VMEM((SBATCH,), jnp.int32),
                      pltpu.VMEM((SBATCH,), jnp.int32),
                      pltpu.VMEM((SBATCH, SW), jnp.float32),
                      pltpu.VMEM((SBATCH, SW), jnp.float32))

    return k


def _make_dim_sc():
    mesh = plsc.VectorSubcoreMesh(core_axis_name="c", subcore_axis_name="s",
                                  num_cores=NC, num_subcores=NS)
    nb = EDGES_PER_TILE // VBATCH

    @functools.partial(
        pl.kernel,
        out_type=jax.ShapeDtypeStruct((NC, N_PAD, SW), jnp.float32),
        mesh=mesh,
        scratch_types=[
            pltpu.VMEM_SHARED((N_PAD, SW), jnp.float32),
            pltpu.SemaphoreType.DMA,
            pltpu.SemaphoreType.DMA,
            pltpu.SemaphoreType.DMA,
        ],
    )
    def k(idx_i_hbm, idx_j_hbm, phi_hbm, w_hbm, vfd_hbm, dd_hbm,
          out_hbm, acc, semL, semG0, semG1):
        c = lax.axis_index("c")
        s = lax.axis_index("s")
        semG = [semG0, semG1]

        def run(*bufs):
            sets = [bufs[:6], bufs[6:]]
            r0 = s * ROWS_PER_TILE
            _zero_and_fill(sets[0][4], acc, r0)
            plsc.subcore_barrier()

            tile_base = (c * NS + s) * EDGES_PER_TILE

            def issue_linear(b, si):
                idxj_v, idxi_v, w_v, pj_v, vf_v, dd_v = sets[si]
                bb = jnp.minimum(b, nb - 1)
                base = tile_base + bb * VBATCH
                pltpu.async_copy(idx_j_hbm.at[pl.ds(base, VBATCH)],
                                 idxj_v, semL)
                pltpu.async_copy(idx_i_hbm.at[pl.ds(base, VBATCH)],
                                 idxi_v, semL)
                pltpu.async_copy(w_hbm.at[pl.ds(base, VBATCH)], w_v, semL)
                pltpu.async_copy(dd_hbm.at[pl.ds(base, VBATCH)],
                                 dd_v.at[pl.ds(0, VBATCH)], semL)

            def wait_linear(si):
                idxj_v, idxi_v, w_v, pj_v, vf_v, dd_v = sets[si]
                pltpu.make_async_copy(
                    idx_j_hbm.at[pl.ds(0, VBATCH)], idxj_v, semL).wait()
                pltpu.make_async_copy(
                    idx_i_hbm.at[pl.ds(0, VBATCH)], idxi_v, semL).wait()
                pltpu.make_async_copy(
                    w_hbm.at[pl.ds(0, VBATCH)], w_v, semL).wait()
                pltpu.make_async_copy(
                    dd_hbm.at[pl.ds(0, VBATCH)],
                    dd_v.at[pl.ds(0, VBATCH)], semL).wait()

            def issue_gather(si):
                idxj_v, idxi_v, w_v, pj_v, vf_v, dd_v = sets[si]
                pltpu.async_copy(phi_hbm.at[idxj_v], pj_v, semG[si])
                pltpu.async_copy(vfd_hbm.at[idxj_v], vf_v, semG[si])

            def wait_gather(si):
                idxj_v, idxi_v, w_v, pj_v, vf_v, dd_v = sets[si]
                pltpu.make_async_copy(phi_hbm.at[idxj_v], pj_v,
                                      semG[si]).wait()
                pltpu.make_async_copy(vfd_hbm.at[idxj_v], vf_v,
                                      semG[si]).wait()

            def compute_scatter(si):
                idxj_v, idxi_v, w_v, pj_v, vf_v, dd_v = sets[si]

                def edge_body(e, c2):
                    db = _lane_bcast(dd_v, e)
                    for kk in range(SW // L):
                        sl = pl.ds(kk * L, L)
                        sv = pl.ds(SW + kk * L, L)
                        vvw = pj_v[e, sl] * w_v[e, sl]
                        vsw = pj_v[e, sv] * w_v[e, sv]
                        vf_v[e, sl] = vf_v[e, sl] * vvw + vsw * db
                    return c2

                lax.fori_loop(0, VBATCH, edge_body, 0)
                pltpu.sync_copy(vf_v, acc.at[idxi_v], add=True)

            issue_linear(0, 0)
            wait_linear(0)
            issue_gather(0)
            issue_linear(1, 1)

            def pair_body(t, carry):
                b0 = 2 * t
                wait_linear(1)
                issue_gather(1)
                wait_gather(0)
                compute_scatter(0)
                issue_linear(b0 + 2, 0)
                wait_linear(0)
                issue_gather(0)
                wait_gather(1)
                compute_scatter(1)
                issue_linear(b0 + 3, 1)
                return carry

            lax.fori_loop(0, nb // 2, pair_body, 0)
            wait_gather(0)
            wait_linear(1)

            plsc.subcore_barrier()
            _flush(acc, out_hbm, c, r0, sets[0][4])

        pl.run_scoped(run,
                      pltpu.VMEM((VBATCH,), jnp.int32),
                      pltpu.VMEM((VBATCH,), jnp.int32),
                      pltpu.VMEM((VBATCH, PW), jnp.float32),
                      pltpu.VMEM((VBATCH, PW), jnp.float32),
                      pltpu.VMEM((VBATCH, SW), jnp.float32),
                      pltpu.VMEM((VBATCH + L,), jnp.float32),
                      pltpu.VMEM((VBATCH,), jnp.int32),
                      pltpu.VMEM((VBATCH,), jnp.int32),
                      pltpu.VMEM((VBATCH, PW), jnp.float32),
                      pltpu.VMEM((VBATCH, PW), jnp.float32),
                      pltpu.VMEM((VBATCH, SW), jnp.float32),
                      pltpu.VMEM((VBATCH + L,), jnp.float32))

    return k


def _combine_body(sf_ref, vfd_ref, ps_ref, p0_ref, p1_ref, p2_ref,
                  os_ref, ov_ref):
    os_ref[...] = sf_ref[...] + ps_ref[0] + ps_ref[1]
    ov_ref[0] = vfd_ref[0] + p0_ref[0] + p0_ref[1]
    ov_ref[1] = vfd_ref[1] + p1_ref[0] + p1_ref[1]
    ov_ref[2] = vfd_ref[2] + p2_ref[0] + p2_ref[1]


def _combine(sf, vfd, ps, p0, p1, p2):
    bm = 400
    grid = (N // bm,)
    return pl.pallas_call(
        _combine_body,
        grid=grid,
        in_specs=[
            pl.BlockSpec((bm, F), lambda i: (i, 0)),
            pl.BlockSpec((3, bm, SW), lambda i: (0, i, 0)),
            pl.BlockSpec((NC, bm, SW), lambda i: (0, i, 0)),
            pl.BlockSpec((NC, bm, SW), lambda i: (0, i, 0)),
            pl.BlockSpec((NC, bm, SW), lambda i: (0, i, 0)),
            pl.BlockSpec((NC, bm, SW), lambda i: (0, i, 0)),
        ],
        out_specs=[
            pl.BlockSpec((bm, F), lambda i: (i, 0)),
            pl.BlockSpec((3, bm, SW), lambda i: (0, i, 0)),
        ],
        out_shape=[
            jax.ShapeDtypeStruct((N, F), jnp.float32),
            jax.ShapeDtypeStruct((3, N, SW), jnp.float32),
        ],
    )(sf, vfd, ps, p0, p1, p2)


_scalar_sc = _make_scalar_sc()
_dim_sc = _make_dim_sc()

_PERM = np.concatenate([
    np.arange(F, 2 * F),
    np.arange(0, F),
    np.arange(2 * F, 3 * F),
]).astype(np.int32)


def _pad_e(x):
    pad = [(0, E_PAD - E)] + [(0, 0)] * (x.ndim - 1)
    return jnp.pad(x, pad)


def kernel(idx_i, idx_j, rel_dir, rel_dist_cut, rbf_features, scalar_features,
           vector_features, W1, b1, W2, b2, Wr, br):
    idx_i = _pad_e(idx_i.astype(jnp.int32))
    idx_j = _pad_e(idx_j.astype(jnp.int32))

    w1t = W1.T
    w2t = W2[_PERM].T
    b2p = b2[_PERM].reshape(1, TF)
    wrt = Wr[_PERM].T
    brp = br[_PERM].reshape(1, TF)
    b1r = b1.reshape(1, F)
    rdc2d = _pad_e(rel_dist_cut.reshape(E, 1))

    phi_s, phi_v = _phi_tables(scalar_features, w1t, b1r, w2t, b2p)
    w_s, w_v = _wedge_tables(_pad_e(rbf_features), wrt, brp, rdc2d)

    vfd = jnp.transpose(vector_features, (2, 0, 1))
    rdp = _pad_e(rel_dir)

    ps = _scalar_sc(idx_i, idx_j, phi_s, w_s)
    pd = [
        _dim_sc(idx_i, idx_j, phi_v, w_v, vfd[d], rdp[:, d])
        for d in range(3)
    ]

    out_s, out_vd = _combine(scalar_features, vfd, ps, pd[0], pd[1], pd[2])
    return out_s, jnp.transpose(out_vd, (1, 2, 0))
